# baseline (speedup 1.0000x reference)
import functools

import jax
import jax.numpy as jnp
from jax import lax
from jax.experimental import pallas as pl
from jax.experimental.pallas import tpu as pltpu

_NUM_EXPERTS = 8
_TOP_K = 2
_ALPHA = 0.001
_BLOCK = 4096


def _gate_kernel(x_ref, w_ref, wt_ref, id_ref, aux_ref, cnt_ref, psum_ref,
                 *, num_tokens):
    step = pl.program_id(0)
    nsteps = pl.num_programs(0)

    @pl.when(step == 0)
    def _init():
        cnt_ref[...] = jnp.zeros_like(cnt_ref)
        psum_ref[...] = jnp.zeros_like(psum_ref)

    logits = lax.dot_general(
        w_ref[...], x_ref[...],
        dimension_numbers=(((1,), (1,)), ((), ())),
        preferred_element_type=jnp.float32,
    )

    b = logits.shape[1]
    eidx = lax.broadcasted_iota(jnp.int32, (_NUM_EXPERTS, b), 0)

    m1 = jnp.max(logits, axis=0, keepdims=True)
    i1 = jnp.min(jnp.where(logits == m1, eidx, _NUM_EXPERTS),
                 axis=0, keepdims=True)

    p = jnp.exp(logits - m1)
    inv_s = 1.0 / jnp.sum(p, axis=0, keepdims=True)

    oh1 = eidx == i1
    pm2 = jnp.max(jnp.where(oh1, -1.0, p), axis=0, keepdims=True)
    i2 = jnp.min(jnp.where(jnp.logical_and(p == pm2, jnp.logical_not(oh1)),
                           eidx, _NUM_EXPERTS), axis=0, keepdims=True)

    inv12 = 1.0 / (1.0 + pm2)
    wt_ref[...] = jnp.concatenate([inv12, pm2 * inv12], axis=0)
    id_ref[...] = jnp.concatenate([i1, i2], axis=0)

    onehots = oh1.astype(jnp.float32) + (eidx == i2).astype(jnp.float32)
    cnt_ref[:, 0:1] += jnp.sum(onehots, axis=1, keepdims=True)
    psum_ref[:, 0:1] += jnp.sum(p * inv_s, axis=1, keepdims=True)

    @pl.when(step == nsteps - 1)
    def _finish():
        scale = _ALPHA * _NUM_EXPERTS / (num_tokens * _TOP_K * num_tokens)
        dot = jnp.sum(cnt_ref[:, 0:1] * psum_ref[:, 0:1], axis=0, keepdims=True)
        aux_ref[...] = dot * scale


def kernel(hidden_states, W):
    n, h = hidden_states.shape
    e = W.shape[0]
    grid = (n // _BLOCK,)

    wt, ids, aux = pl.pallas_call(
        functools.partial(_gate_kernel, num_tokens=n),
        grid=grid,
        in_specs=[
            pl.BlockSpec((_BLOCK, h), lambda i: (i, 0)),
            pl.BlockSpec((e, h), lambda i: (0, 0)),
        ],
        out_specs=[
            pl.BlockSpec((_TOP_K, _BLOCK), lambda i: (0, i)),
            pl.BlockSpec((_TOP_K, _BLOCK), lambda i: (0, i)),
            pl.BlockSpec((1, 1), lambda i: (0, 0)),
        ],
        out_shape=[
            jax.ShapeDtypeStruct((_TOP_K, n), jnp.float32),
            jax.ShapeDtypeStruct((_TOP_K, n), jnp.int32),
            jax.ShapeDtypeStruct((1, 1), jnp.float32),
        ],
        scratch_shapes=[
            pltpu.VMEM((_NUM_EXPERTS, 128), jnp.float32),
            pltpu.VMEM((_NUM_EXPERTS, 128), jnp.float32),
        ],
        compiler_params=pltpu.CompilerParams(
            dimension_semantics=("arbitrary",),
        ),
    )(hidden_states, W)

    return wt.T, ids.T, aux[0, 0]
